# baseline (speedup 1.0000x reference)
import functools

import jax
import jax.numpy as jnp
from jax import lax
from jax.experimental import pallas as pl
from jax.experimental.pallas import tpu as pltpu
from jax.experimental.pallas import tpu_sc as plsc

_BK = 2560
_KCH = 80


def _sc_gather(table, idx3):
    NW, n_chunks, K = idx3.shape
    D = table.shape[1]
    dt = table.dtype
    total = NW * n_chunks * K
    info = plsc.get_sparse_core_info()
    NC = info.num_cores
    per_w = total // NW
    n2 = n_chunks // 2
    mesh = plsc.VectorSubcoreMesh(core_axis_name="c", subcore_axis_name="s")

    @functools.partial(
        pl.kernel,
        mesh=mesh,
        out_type=jax.ShapeDtypeStruct((total, D), dt),
        scratch_types=[
            pltpu.VMEM((n_chunks, K), jnp.int32),
            pltpu.VMEM((K, D), dt),
            pltpu.VMEM((K, D), dt),
            pltpu.SemaphoreType.DMA,
            pltpu.SemaphoreType.DMA,
            pltpu.SemaphoreType.DMA,
            pltpu.SemaphoreType.DMA,
        ],
    )
    def gather_k(table_hbm, idx_hbm, out_hbm, idx_all, r0, r1,
                 sg0, sg1, so0, so1):
        wid = lax.axis_index("s") * NC + lax.axis_index("c")
        base = wid * per_w
        pltpu.sync_copy(idx_hbm.at[wid], idx_all)

        def start_g(i, buf, sem):
            pltpu.async_copy(table_hbm.at[idx_all.at[i]], buf, sem)

        def wait_g(buf, sem):
            pltpu.make_async_copy(table_hbm.at[idx_all.at[0]], buf, sem).wait()

        def start_o(i, buf, sem):
            off = pl.multiple_of(base + i * K, 8)
            pltpu.async_copy(buf, out_hbm.at[pl.ds(off, K)], sem)

        def wait_o(buf, sem):
            pltpu.make_async_copy(buf, out_hbm.at[pl.ds(base, K)], sem).wait()

        start_g(0, r0, sg0)

        def body(j, carry):
            @pl.when(j > 0)
            def _():
                wait_o(r1, so1)
            start_g(2 * j + 1, r1, sg1)
            wait_g(r0, sg0)
            start_o(2 * j, r0, so0)
            wait_g(r1, sg1)
            wait_o(r0, so0)

            @pl.when(j < n2 - 1)
            def _():
                start_g(2 * j + 2, r0, sg0)
            start_o(2 * j + 1, r1, so1)
            return carry

        lax.fori_loop(0, n2, body, 0)
        wait_o(r1, so1)

    return gather_k(table, idx3)


def _sc_scatter_add(payload, idx4, zeros, n_nodes):
    NW, n_ph, PH, K = idx4.shape
    D = payload.shape[1]
    total = NW * n_ph * PH * K
    n_chunks = n_ph * PH
    info = plsc.get_sparse_core_info()
    NC, NS = info.num_cores, info.num_subcores
    per_w = total // NW
    rows_per_s = (n_nodes // NS) & ~7
    tail = n_nodes - rows_per_s * NS
    mesh = plsc.VectorSubcoreMesh(core_axis_name="c", subcore_axis_name="s")

    @functools.partial(
        pl.kernel,
        mesh=mesh,
        out_type=jax.ShapeDtypeStruct((NC, n_nodes, D), jnp.float32),
        scratch_types=[
            pltpu.VMEM((PH, K), jnp.int32),
            pltpu.VMEM((K, D), jnp.float32),
            pltpu.VMEM((K, D), jnp.float32),
            pltpu.VMEM_SHARED((n_nodes, D), jnp.float32),
            pltpu.SemaphoreType.DMA,
            pltpu.SemaphoreType.DMA,
        ],
    )
    def scatter_k(p_hbm, idx_hbm, z_hbm, out_hbm, idx_ph, r0, r1, acc,
                  sp0, sp1):
        c = lax.axis_index("c")
        s = lax.axis_index("s")

        @pl.when(s == 0)
        def _init():
            pltpu.sync_copy(z_hbm, acc)

        wid = s * NC + c
        base = wid * per_w
        plsc.subcore_barrier()

        def start_p(i, buf, sem):
            off = pl.multiple_of(base + i * K, 8)
            pltpu.async_copy(p_hbm.at[pl.ds(off, K)], buf, sem)

        def wait_p(buf, sem):
            pltpu.make_async_copy(p_hbm.at[pl.ds(base, K)], buf, sem).wait()

        start_p(0, r0, sp0)

        def phase(p, carry):
            pltpu.sync_copy(idx_hbm.at[wid, p], idx_ph)

            def body(jj, c2):
                g0 = p * PH + 2 * jj
                start_p(g0 + 1, r1, sp1)
                wait_p(r0, sp0)
                pltpu.sync_copy(r0, acc.at[idx_ph.at[2 * jj]], add=True)

                @pl.when(g0 + 2 < n_chunks)
                def _():
                    start_p(g0 + 2, r0, sp0)
                wait_p(r1, sp1)
                pltpu.sync_copy(r1, acc.at[idx_ph.at[2 * jj + 1]], add=True)
                return c2

            lax.fori_loop(0, PH // 2, body, carry)
            return carry

        lax.fori_loop(0, n_ph, phase, 0)
        plsc.subcore_barrier()
        r0o = pl.multiple_of(s * rows_per_s, 8)
        pltpu.sync_copy(acc.at[pl.ds(r0o, rows_per_s)],
                        out_hbm.at[c, pl.ds(r0o, rows_per_s)])
        if tail:
            @pl.when(s == NS - 1)
            def _tail():
                t0 = rows_per_s * NS
                pltpu.sync_copy(acc.at[pl.ds(t0, tail)],
                                out_hbm.at[c, pl.ds(t0, tail)])

    return scatter_k(payload, idx4, zeros)


def _tc_pass1(gpair, epair, lvl1_W, lift_W1, eps2):
    Ep, two_h = gpair.shape
    H = two_h // 2
    grid = Ep // _BK
    n_entries = jnp.float32(2 * Ep)
    del n_entries

    def body(g_ref, er_ref, w_ref, u_ref, eps_ref, z1_ref, z2_ref,
             st1_ref, st2_ref):
        f32, bf16 = jnp.float32, jnp.bfloat16
        g = g_ref[...]
        er = er_ref[...]
        ge, go = g[:, :H], g[:, H:]
        ere, ero = er[:, :H], er[:, H:]
        psum = ge + go
        pmean = 0.5 * (ere + ero)
        W = w_ref[...].astype(bf16)
        Wa, Wb, Wc = W[:H], W[H:2 * H], W[2 * H:]

        def bdot(x, w):
            return jnp.dot(x.astype(bf16), w, preferred_element_type=f32)

        sb = bdot(psum, Wb)
        z1e = bdot(ge, Wa) + sb + bdot(ere, Wc)
        z1o = bdot(go, Wa) + sb + bdot(ero, Wc)
        a = 1.0 + eps_ref[0, 0]
        U = u_ref[...].astype(bf16)
        Ua, Ub = U[:H], U[H:]
        tb = bdot(a * pmean + psum, Ub)
        z2e = bdot(a * ere + ge, Ua) + tb
        z2o = bdot(a * ero + go, Ua) + tb
        z1_ref[...] = jnp.concatenate([z1e, z1o], axis=1).astype(bf16)
        z2_ref[...] = jnp.concatenate([z2e, z2o], axis=1).astype(bf16)
        s1 = (jnp.sum(z1e, 0) + jnp.sum(z1o, 0))[None, :]
        q1 = (jnp.sum(z1e * z1e, 0) + jnp.sum(z1o * z1o, 0))[None, :]
        s2 = (jnp.sum(z2e, 0) + jnp.sum(z2o, 0))[None, :]
        q2 = (jnp.sum(z2e * z2e, 0) + jnp.sum(z2o * z2o, 0))[None, :]
        new1 = jnp.concatenate([s1, q1, jnp.zeros((6, H), f32)], axis=0)
        new2 = jnp.concatenate([s2, q2, jnp.zeros((6, 2 * H), f32)], axis=0)
        first = pl.program_id(0) == 0
        st1_ref[...] = jnp.where(first, new1, st1_ref[...] + new1)
        st2_ref[...] = jnp.where(first, new2, st2_ref[...] + new2)

    return pl.pallas_call(
        body,
        grid=(grid,),
        in_specs=[
            pl.BlockSpec((_BK, 2 * H), lambda i: (i, 0)),
            pl.BlockSpec((_BK, 2 * H), lambda i: (i, 0)),
            pl.BlockSpec((3 * H, H), lambda i: (0, 0)),
            pl.BlockSpec((2 * H, 2 * H), lambda i: (0, 0)),
            pl.BlockSpec((1, 1), lambda i: (0, 0)),
        ],
        out_specs=[
            pl.BlockSpec((_BK, 2 * H), lambda i: (i, 0)),
            pl.BlockSpec((_BK, 4 * H), lambda i: (i, 0)),
            pl.BlockSpec((8, H), lambda i: (0, 0)),
            pl.BlockSpec((8, 2 * H), lambda i: (0, 0)),
        ],
        out_shape=[
            jax.ShapeDtypeStruct((Ep, 2 * H), jnp.bfloat16),
            jax.ShapeDtypeStruct((Ep, 4 * H), jnp.bfloat16),
            jax.ShapeDtypeStruct((8, H), jnp.float32),
            jax.ShapeDtypeStruct((8, 2 * H), jnp.float32),
        ],
    )(gpair, epair, lvl1_W, lift_W1, eps2)


def _tc_pass2(z1, z2, st1, st2, lvl1_g, lvl1_b, lift_g1, lift_b1, lift_W2,
              eps1_2):
    Ep, two_h = z1.shape
    H = two_h // 2
    grid = Ep // _BK
    inv_n = 1.0 / float(2 * Ep)

    def body(z1_ref, z2_ref, st1_ref, st2_ref, g1_ref, b1_ref, lg_ref,
             lb_ref, w2_ref, eps_ref, p_ref, z3_ref, st3_ref):
        f32 = jnp.float32
        s1 = st1_ref[0, :]
        q1 = st1_ref[1, :]
        m1 = s1 * inv_n
        v1 = q1 * inv_n - m1 * m1
        sc1 = g1_ref[0, :] * lax.rsqrt(v1 + 1e-5)
        of1 = b1_ref[0, :] - m1 * sc1
        z1b = z1_ref[...].astype(f32)
        h1e = jnp.maximum(z1b[:, :H] * sc1 + of1, 0.0)
        h1o = jnp.maximum(z1b[:, H:] * sc1 + of1, 0.0)
        w = 2.0 + eps_ref[0, 0]
        p_ref[...] = jnp.concatenate([w * h1e + h1o, h1e + w * h1o], axis=1)
        s2 = st2_ref[0, :]
        q2 = st2_ref[1, :]
        m2 = s2 * inv_n
        v2 = q2 * inv_n - m2 * m2
        sc2 = lg_ref[0, :] * lax.rsqrt(v2 + 1e-5)
        of2 = lb_ref[0, :] - m2 * sc2
        z2b = z2_ref[...].astype(f32)
        ue = jnp.maximum(z2b[:, :2 * H] * sc2 + of2, 0.0)
        uo = jnp.maximum(z2b[:, 2 * H:] * sc2 + of2, 0.0)
        W2 = w2_ref[...].astype(jnp.bfloat16)
        z3e = jnp.dot(ue.astype(jnp.bfloat16), W2, preferred_element_type=f32)
        z3o = jnp.dot(uo.astype(jnp.bfloat16), W2, preferred_element_type=f32)
        z3_ref[...] = jnp.concatenate([z3e, z3o], axis=1).astype(jnp.bfloat16)
        s3 = (jnp.sum(z3e, 0) + jnp.sum(z3o, 0))[None, :]
        q3 = (jnp.sum(z3e * z3e, 0) + jnp.sum(z3o * z3o, 0))[None, :]
        new3 = jnp.concatenate([s3, q3, jnp.zeros((6, H), f32)], axis=0)
        first = pl.program_id(0) == 0
        st3_ref[...] = jnp.where(first, new3, st3_ref[...] + new3)

    return pl.pallas_call(
        body,
        grid=(grid,),
        in_specs=[
            pl.BlockSpec((_BK, 2 * H), lambda i: (i, 0)),
            pl.BlockSpec((_BK, 4 * H), lambda i: (i, 0)),
            pl.BlockSpec((8, H), lambda i: (0, 0)),
            pl.BlockSpec((8, 2 * H), lambda i: (0, 0)),
            pl.BlockSpec((1, H), lambda i: (0, 0)),
            pl.BlockSpec((1, H), lambda i: (0, 0)),
            pl.BlockSpec((1, 2 * H), lambda i: (0, 0)),
            pl.BlockSpec((1, 2 * H), lambda i: (0, 0)),
            pl.BlockSpec((2 * H, H), lambda i: (0, 0)),
            pl.BlockSpec((1, 1), lambda i: (0, 0)),
        ],
        out_specs=[
            pl.BlockSpec((_BK, 2 * H), lambda i: (i, 0)),
            pl.BlockSpec((_BK, 2 * H), lambda i: (i, 0)),
            pl.BlockSpec((8, H), lambda i: (0, 0)),
        ],
        out_shape=[
            jax.ShapeDtypeStruct((Ep, 2 * H), jnp.float32),
            jax.ShapeDtypeStruct((Ep, 2 * H), jnp.bfloat16),
            jax.ShapeDtypeStruct((8, H), jnp.float32),
        ],
    )(z1, z2, st1, st2, lvl1_g, lvl1_b, lift_g1, lift_b1, lift_W2, eps1_2)


def _tc_pass3_node(z3, st3, lift_g2, lift_b2, node_rep, acc2, lvl2_W1,
                   lvl2_g1, lvl2_b1, lvl2_W2, lvl2_g2, lvl2_b2, eps1_1):
    Ep, two_h = z3.shape
    H = two_h // 2
    N = node_rep.shape[0]
    grid = Ep // _BK
    inv_n = 1.0 / float(2 * Ep)

    def body(z3_ref, st3_ref, g_ref, b_ref, nr_ref, acc_ref, w1_ref,
             g1_ref, b1_ref, w2_ref, g2_ref, b2_ref, eps_ref,
             out_ref, nout_ref):
        f32 = jnp.float32
        i = pl.program_id(0)

        @pl.when(i < grid)
        def _edges():
            s3 = st3_ref[0, :]
            q3 = st3_ref[1, :]
            m3 = s3 * inv_n
            v3 = q3 * inv_n - m3 * m3
            sc3 = g_ref[0, :] * lax.rsqrt(v3 + 1e-5)
            of3 = b_ref[0, :] - m3 * sc3
            sc = jnp.concatenate([sc3, sc3])
            of = jnp.concatenate([of3, of3])
            out_ref[...] = jnp.maximum(z3_ref[...].astype(f32) * sc + of, 0.0)

        @pl.when(i == grid)
        def _nodes():
            acc = acc_ref[...]
            node_in = (1.0 + eps_ref[0, 0]) * nr_ref[...] + acc[:N] + acc[N:]
            z = jnp.dot(node_in, w1_ref[...], preferred_element_type=f32)
            m = jnp.mean(z, axis=0)
            v = jnp.mean(z * z, axis=0) - m * m
            sc = g1_ref[0, :] * lax.rsqrt(v + 1e-5)
            t = jnp.maximum(z * sc + (b1_ref[0, :] - m * sc), 0.0)
            z2 = jnp.dot(t, w2_ref[...], preferred_element_type=f32)
            m2 = jnp.mean(z2, axis=0)
            v2 = jnp.mean(z2 * z2, axis=0) - m2 * m2
            sc2 = g2_ref[0, :] * lax.rsqrt(v2 + 1e-5)
            nout_ref[...] = jnp.maximum(z2 * sc2 + (b2_ref[0, :] - m2 * sc2),
                                        0.0)

    ce = lambda i: (jnp.minimum(i, grid - 1), 0)
    c0 = lambda i: (0, 0)
    return pl.pallas_call(
        body,
        grid=(grid + 1,),
        in_specs=[
            pl.BlockSpec((_BK, 2 * H), ce),
            pl.BlockSpec((8, H), c0),
            pl.BlockSpec((1, H), c0),
            pl.BlockSpec((1, H), c0),
            pl.BlockSpec((N, H), c0),
            pl.BlockSpec((2 * N, H), c0),
            pl.BlockSpec((H, 2 * H), c0),
            pl.BlockSpec((1, 2 * H), c0),
            pl.BlockSpec((1, 2 * H), c0),
            pl.BlockSpec((2 * H, H), c0),
            pl.BlockSpec((1, H), c0),
            pl.BlockSpec((1, H), c0),
            pl.BlockSpec((1, 1), c0),
        ],
        out_specs=[
            pl.BlockSpec((_BK, 2 * H), ce),
            pl.BlockSpec((N, H), c0),
        ],
        out_shape=[
            jax.ShapeDtypeStruct((Ep, 2 * H), jnp.float32),
            jax.ShapeDtypeStruct((N, H), jnp.float32),
        ],
    )(z3, st3, lift_g2, lift_b2, node_rep, acc2, lvl2_W1, lvl2_g1, lvl2_b1,
      lvl2_W2, lvl2_g2, lvl2_b2, eps1_1)


def kernel(node_rep, edge_rep, edge_index, lift_W1, lift_g1, lift_b1,
           lift_W2, lift_g2, lift_b2, lvl1_W, lvl1_g, lvl1_b, lvl2_W1,
           lvl2_g1, lvl2_b1, lvl2_W2, lvl2_g2, lvl2_b2, eps1_1, eps1_2,
           eps2):
    N, H = node_rep.shape
    E = edge_index.shape[1]
    entry_nodes = edge_index.T.reshape(-1)
    info = plsc.get_sparse_core_info()
    NW = info.num_cores * info.num_subcores
    idx3 = entry_nodes.reshape(NW, (2 * E) // (NW * _KCH), _KCH)
    n_chunks = (2 * E) // (NW * _KCH)
    n_ph = 5
    idx4 = entry_nodes.reshape(NW, n_ph, n_chunks // n_ph, _KCH)

    gA = _sc_gather(node_rep, idx3)
    gpair = gA.reshape(E, 2 * H)
    epair = edge_rep.reshape(E, 2 * H)

    r2 = lambda x: x.reshape(1, -1)
    s2 = lambda x: x.reshape(1, 1)

    z1, z2, st1, st2 = _tc_pass1(gpair, epair, lvl1_W, lift_W1, s2(eps2))
    P, z3, st3 = _tc_pass2(z1, z2, st1, st2, r2(lvl1_g), r2(lvl1_b),
                           r2(lift_g1), r2(lift_b1), lift_W2, s2(eps1_2))

    zeros = jnp.zeros((N, H), jnp.float32)
    acc = _sc_scatter_add(P.reshape(2 * E, H), idx4, zeros, N)
    acc2 = acc.reshape(2 * N, H)

    edge_out, node_out = _tc_pass3_node(
        z3, st3, r2(lift_g2), r2(lift_b2), node_rep, acc2, lvl2_W1,
        r2(lvl2_g1), r2(lvl2_b1), lvl2_W2, r2(lvl2_g2), r2(lvl2_b2),
        s2(eps1_1))
    return node_out, edge_out.reshape(2 * E, H)


# speedup vs baseline: 1.0668x; 1.0668x over previous
import functools

import jax
import jax.numpy as jnp
from jax import lax
from jax.experimental import pallas as pl
from jax.experimental.pallas import tpu as pltpu
from jax.experimental.pallas import tpu_sc as plsc

_BK = 3200
_KCH = 80


def _sc_gather(table, idx3):
    NW, n_chunks, K = idx3.shape
    D = table.shape[1]
    dt = table.dtype
    total = NW * n_chunks * K
    info = plsc.get_sparse_core_info()
    NC = info.num_cores
    per_w = total // NW
    n2 = n_chunks // 2
    mesh = plsc.VectorSubcoreMesh(core_axis_name="c", subcore_axis_name="s")

    @functools.partial(
        pl.kernel,
        mesh=mesh,
        out_type=jax.ShapeDtypeStruct((total, D), dt),
        scratch_types=[
            pltpu.VMEM((n_chunks, K), jnp.int32),
            pltpu.VMEM((K, D), dt),
            pltpu.VMEM((K, D), dt),
            pltpu.SemaphoreType.DMA,
            pltpu.SemaphoreType.DMA,
            pltpu.SemaphoreType.DMA,
            pltpu.SemaphoreType.DMA,
        ],
    )
    def gather_k(table_hbm, idx_hbm, out_hbm, idx_all, r0, r1,
                 sg0, sg1, so0, so1):
        wid = lax.axis_index("s") * NC + lax.axis_index("c")
        base = wid * per_w
        pltpu.sync_copy(idx_hbm.at[wid], idx_all)

        def start_g(i, buf, sem):
            pltpu.async_copy(table_hbm.at[idx_all.at[i]], buf, sem)

        def wait_g(buf, sem):
            pltpu.make_async_copy(table_hbm.at[idx_all.at[0]], buf, sem).wait()

        def start_o(i, buf, sem):
            off = pl.multiple_of(base + i * K, 8)
            pltpu.async_copy(buf, out_hbm.at[pl.ds(off, K)], sem)

        def wait_o(buf, sem):
            pltpu.make_async_copy(buf, out_hbm.at[pl.ds(base, K)], sem).wait()

        start_g(0, r0, sg0)

        def body(j, carry):
            @pl.when(j > 0)
            def _():
                wait_o(r1, so1)
            start_g(2 * j + 1, r1, sg1)
            wait_g(r0, sg0)
            start_o(2 * j, r0, so0)
            wait_g(r1, sg1)
            wait_o(r0, so0)

            @pl.when(j < n2 - 1)
            def _():
                start_g(2 * j + 2, r0, sg0)
            start_o(2 * j + 1, r1, so1)
            return carry

        lax.fori_loop(0, n2, body, 0)
        wait_o(r1, so1)

    return gather_k(table, idx3)


def _sc_scatter_add(payload, idx4, zeros, n_nodes):
    NW, n_ph, PH, K = idx4.shape
    D = payload.shape[1]
    total = NW * n_ph * PH * K
    n_chunks = n_ph * PH
    info = plsc.get_sparse_core_info()
    NC, NS = info.num_cores, info.num_subcores
    per_w = total // NW
    rows_per_s = (n_nodes // NS) & ~7
    tail = n_nodes - rows_per_s * NS
    mesh = plsc.VectorSubcoreMesh(core_axis_name="c", subcore_axis_name="s")

    @functools.partial(
        pl.kernel,
        mesh=mesh,
        out_type=jax.ShapeDtypeStruct((NC, n_nodes, D), jnp.float32),
        scratch_types=[
            pltpu.VMEM((PH, K), jnp.int32),
            pltpu.VMEM((K, D), jnp.float32),
            pltpu.VMEM((K, D), jnp.float32),
            pltpu.VMEM_SHARED((n_nodes, D), jnp.float32),
            pltpu.SemaphoreType.DMA,
            pltpu.SemaphoreType.DMA,
        ],
    )
    def scatter_k(p_hbm, idx_hbm, z_hbm, out_hbm, idx_ph, r0, r1, acc,
                  sp0, sp1):
        c = lax.axis_index("c")
        s = lax.axis_index("s")

        @pl.when(s == 0)
        def _init():
            pltpu.sync_copy(z_hbm, acc)

        wid = s * NC + c
        base = wid * per_w
        plsc.subcore_barrier()

        def start_p(i, buf, sem):
            off = pl.multiple_of(base + i * K, 8)
            pltpu.async_copy(p_hbm.at[pl.ds(off, K)], buf, sem)

        def wait_p(buf, sem):
            pltpu.make_async_copy(p_hbm.at[pl.ds(base, K)], buf, sem).wait()

        start_p(0, r0, sp0)

        def phase(p, carry):
            pltpu.sync_copy(idx_hbm.at[wid, p], idx_ph)

            def body(jj, c2):
                g0 = p * PH + 2 * jj
                start_p(g0 + 1, r1, sp1)
                wait_p(r0, sp0)
                pltpu.sync_copy(r0, acc.at[idx_ph.at[2 * jj]], add=True)

                @pl.when(g0 + 2 < n_chunks)
                def _():
                    start_p(g0 + 2, r0, sp0)
                wait_p(r1, sp1)
                pltpu.sync_copy(r1, acc.at[idx_ph.at[2 * jj + 1]], add=True)
                return c2

            lax.fori_loop(0, PH // 2, body, carry)
            return carry

        lax.fori_loop(0, n_ph, phase, 0)
        plsc.subcore_barrier()
        r0o = pl.multiple_of(s * rows_per_s, 8)
        pltpu.sync_copy(acc.at[pl.ds(r0o, rows_per_s)],
                        out_hbm.at[c, pl.ds(r0o, rows_per_s)])
        if tail:
            @pl.when(s == NS - 1)
            def _tail():
                t0 = rows_per_s * NS
                pltpu.sync_copy(acc.at[pl.ds(t0, tail)],
                                out_hbm.at[c, pl.ds(t0, tail)])

    return scatter_k(payload, idx4, zeros)


def _tc_pass1(gpair, epair, lvl1_W, lift_W1, eps2):
    Ep, two_h = gpair.shape
    H = two_h // 2
    grid = Ep // _BK
    n_entries = jnp.float32(2 * Ep)
    del n_entries

    def body(g_ref, er_ref, w_ref, u_ref, eps_ref, z1_ref, z2_ref,
             st1_ref, st2_ref):
        f32, bf16 = jnp.float32, jnp.bfloat16
        g = g_ref[...]
        er = er_ref[...]
        ge, go = g[:, :H], g[:, H:]
        ere, ero = er[:, :H], er[:, H:]
        psum = ge + go
        pmean = 0.5 * (ere + ero)
        W = w_ref[...].astype(bf16)
        Wa, Wb, Wc = W[:H], W[H:2 * H], W[2 * H:]

        def bdot(x, w):
            return jnp.dot(x.astype(bf16), w, preferred_element_type=f32)

        sb = bdot(psum, Wb)
        z1e = bdot(ge, Wa) + sb + bdot(ere, Wc)
        z1o = bdot(go, Wa) + sb + bdot(ero, Wc)
        a = 1.0 + eps_ref[0, 0]
        U = u_ref[...].astype(bf16)
        Ua, Ub = U[:H], U[H:]
        tb = bdot(a * pmean + psum, Ub)
        z2e = bdot(a * ere + ge, Ua) + tb
        z2o = bdot(a * ero + go, Ua) + tb
        z1_ref[...] = jnp.concatenate([z1e, z1o], axis=1).astype(bf16)
        z2_ref[...] = jnp.concatenate([z2e, z2o], axis=1).astype(bf16)
        s1 = (jnp.sum(z1e, 0) + jnp.sum(z1o, 0))[None, :]
        q1 = (jnp.sum(z1e * z1e, 0) + jnp.sum(z1o * z1o, 0))[None, :]
        s2 = (jnp.sum(z2e, 0) + jnp.sum(z2o, 0))[None, :]
        q2 = (jnp.sum(z2e * z2e, 0) + jnp.sum(z2o * z2o, 0))[None, :]
        new1 = jnp.concatenate([s1, q1, jnp.zeros((6, H), f32)], axis=0)
        new2 = jnp.concatenate([s2, q2, jnp.zeros((6, 2 * H), f32)], axis=0)
        first = pl.program_id(0) == 0
        st1_ref[...] = jnp.where(first, new1, st1_ref[...] + new1)
        st2_ref[...] = jnp.where(first, new2, st2_ref[...] + new2)

    return pl.pallas_call(
        body,
        grid=(grid,),
        in_specs=[
            pl.BlockSpec((_BK, 2 * H), lambda i: (i, 0)),
            pl.BlockSpec((_BK, 2 * H), lambda i: (i, 0)),
            pl.BlockSpec((3 * H, H), lambda i: (0, 0)),
            pl.BlockSpec((2 * H, 2 * H), lambda i: (0, 0)),
            pl.BlockSpec((1, 1), lambda i: (0, 0)),
        ],
        out_specs=[
            pl.BlockSpec((_BK, 2 * H), lambda i: (i, 0)),
            pl.BlockSpec((_BK, 4 * H), lambda i: (i, 0)),
            pl.BlockSpec((8, H), lambda i: (0, 0)),
            pl.BlockSpec((8, 2 * H), lambda i: (0, 0)),
        ],
        out_shape=[
            jax.ShapeDtypeStruct((Ep, 2 * H), jnp.bfloat16),
            jax.ShapeDtypeStruct((Ep, 4 * H), jnp.bfloat16),
            jax.ShapeDtypeStruct((8, H), jnp.float32),
            jax.ShapeDtypeStruct((8, 2 * H), jnp.float32),
        ],
    )(gpair, epair, lvl1_W, lift_W1, eps2)


def _tc_pass2(z1, z2, st1, st2, lvl1_g, lvl1_b, lift_g1, lift_b1, lift_W2,
              eps1_2):
    Ep, two_h = z1.shape
    H = two_h // 2
    grid = Ep // _BK
    inv_n = 1.0 / float(2 * Ep)

    def body(z1_ref, z2_ref, st1_ref, st2_ref, g1_ref, b1_ref, lg_ref,
             lb_ref, w2_ref, eps_ref, p_ref, z3_ref, st3_ref):
        f32 = jnp.float32
        s1 = st1_ref[0, :]
        q1 = st1_ref[1, :]
        m1 = s1 * inv_n
        v1 = q1 * inv_n - m1 * m1
        sc1 = g1_ref[0, :] * lax.rsqrt(v1 + 1e-5)
        of1 = b1_ref[0, :] - m1 * sc1
        z1b = z1_ref[...].astype(f32)
        h1e = jnp.maximum(z1b[:, :H] * sc1 + of1, 0.0)
        h1o = jnp.maximum(z1b[:, H:] * sc1 + of1, 0.0)
        w = 2.0 + eps_ref[0, 0]
        p_ref[...] = jnp.concatenate([w * h1e + h1o, h1e + w * h1o], axis=1)
        s2 = st2_ref[0, :]
        q2 = st2_ref[1, :]
        m2 = s2 * inv_n
        v2 = q2 * inv_n - m2 * m2
        sc2 = lg_ref[0, :] * lax.rsqrt(v2 + 1e-5)
        of2 = lb_ref[0, :] - m2 * sc2
        z2b = z2_ref[...].astype(f32)
        ue = jnp.maximum(z2b[:, :2 * H] * sc2 + of2, 0.0)
        uo = jnp.maximum(z2b[:, 2 * H:] * sc2 + of2, 0.0)
        W2 = w2_ref[...].astype(jnp.bfloat16)
        z3e = jnp.dot(ue.astype(jnp.bfloat16), W2, preferred_element_type=f32)
        z3o = jnp.dot(uo.astype(jnp.bfloat16), W2, preferred_element_type=f32)
        z3_ref[...] = jnp.concatenate([z3e, z3o], axis=1).astype(jnp.bfloat16)
        s3 = (jnp.sum(z3e, 0) + jnp.sum(z3o, 0))[None, :]
        q3 = (jnp.sum(z3e * z3e, 0) + jnp.sum(z3o * z3o, 0))[None, :]
        new3 = jnp.concatenate([s3, q3, jnp.zeros((6, H), f32)], axis=0)
        first = pl.program_id(0) == 0
        st3_ref[...] = jnp.where(first, new3, st3_ref[...] + new3)

    return pl.pallas_call(
        body,
        grid=(grid,),
        in_specs=[
            pl.BlockSpec((_BK, 2 * H), lambda i: (i, 0)),
            pl.BlockSpec((_BK, 4 * H), lambda i: (i, 0)),
            pl.BlockSpec((8, H), lambda i: (0, 0)),
            pl.BlockSpec((8, 2 * H), lambda i: (0, 0)),
            pl.BlockSpec((1, H), lambda i: (0, 0)),
            pl.BlockSpec((1, H), lambda i: (0, 0)),
            pl.BlockSpec((1, 2 * H), lambda i: (0, 0)),
            pl.BlockSpec((1, 2 * H), lambda i: (0, 0)),
            pl.BlockSpec((2 * H, H), lambda i: (0, 0)),
            pl.BlockSpec((1, 1), lambda i: (0, 0)),
        ],
        out_specs=[
            pl.BlockSpec((_BK, 2 * H), lambda i: (i, 0)),
            pl.BlockSpec((_BK, 2 * H), lambda i: (i, 0)),
            pl.BlockSpec((8, H), lambda i: (0, 0)),
        ],
        out_shape=[
            jax.ShapeDtypeStruct((Ep, 2 * H), jnp.float32),
            jax.ShapeDtypeStruct((Ep, 2 * H), jnp.bfloat16),
            jax.ShapeDtypeStruct((8, H), jnp.float32),
        ],
    )(z1, z2, st1, st2, lvl1_g, lvl1_b, lift_g1, lift_b1, lift_W2, eps1_2)


def _tc_pass3(z3, st3, lift_g2, lift_b2):
    Ep, two_h = z3.shape
    H = two_h // 2
    grid = Ep // _BK
    inv_n = 1.0 / float(2 * Ep)

    def body(z3_ref, st3_ref, g_ref, b_ref, out_ref):
        s3 = st3_ref[0, :]
        q3 = st3_ref[1, :]
        m3 = s3 * inv_n
        v3 = q3 * inv_n - m3 * m3
        sc3 = g_ref[0, :] * lax.rsqrt(v3 + 1e-5)
        of3 = b_ref[0, :] - m3 * sc3
        sc = jnp.concatenate([sc3, sc3])
        of = jnp.concatenate([of3, of3])
        out_ref[...] = jnp.maximum(z3_ref[...].astype(jnp.float32) * sc + of,
                                   0.0)

    return pl.pallas_call(
        body,
        grid=(grid,),
        in_specs=[
            pl.BlockSpec((_BK, 2 * H), lambda i: (i, 0)),
            pl.BlockSpec((8, H), lambda i: (0, 0)),
            pl.BlockSpec((1, H), lambda i: (0, 0)),
            pl.BlockSpec((1, H), lambda i: (0, 0)),
        ],
        out_specs=pl.BlockSpec((_BK, 2 * H), lambda i: (i, 0)),
        out_shape=jax.ShapeDtypeStruct((Ep, 2 * H), jnp.float32),
    )(z3, st3, lift_g2, lift_b2)


def _tc_node(node_rep, acc2, lvl2_W1, lvl2_g1, lvl2_b1, lvl2_W2, lvl2_g2,
             lvl2_b2, eps1_1):
    N, H = node_rep.shape

    def body(nr_ref, acc_ref, w1_ref, g1_ref, b1_ref, w2_ref, g2_ref,
             b2_ref, eps_ref, out_ref):
        f32 = jnp.float32
        acc = acc_ref[...]
        node_in = (1.0 + eps_ref[0, 0]) * nr_ref[...] + acc[:N] + acc[N:]
        z = jnp.dot(node_in, w1_ref[...], preferred_element_type=f32)
        m = jnp.mean(z, axis=0)
        v = jnp.mean(z * z, axis=0) - m * m
        sc = g1_ref[0, :] * lax.rsqrt(v + 1e-5)
        t = jnp.maximum(z * sc + (b1_ref[0, :] - m * sc), 0.0)
        z2 = jnp.dot(t, w2_ref[...], preferred_element_type=f32)
        m2 = jnp.mean(z2, axis=0)
        v2 = jnp.mean(z2 * z2, axis=0) - m2 * m2
        sc2 = g2_ref[0, :] * lax.rsqrt(v2 + 1e-5)
        out_ref[...] = jnp.maximum(z2 * sc2 + (b2_ref[0, :] - m2 * sc2), 0.0)

    return pl.pallas_call(
        body,
        out_shape=jax.ShapeDtypeStruct((N, H), jnp.float32),
    )(node_rep, acc2, lvl2_W1, lvl2_g1, lvl2_b1, lvl2_W2, lvl2_g2, lvl2_b2,
      eps1_1)


def kernel(node_rep, edge_rep, edge_index, lift_W1, lift_g1, lift_b1,
           lift_W2, lift_g2, lift_b2, lvl1_W, lvl1_g, lvl1_b, lvl2_W1,
           lvl2_g1, lvl2_b1, lvl2_W2, lvl2_g2, lvl2_b2, eps1_1, eps1_2,
           eps2):
    N, H = node_rep.shape
    E = edge_index.shape[1]
    entry_nodes = edge_index.T.reshape(-1)
    info = plsc.get_sparse_core_info()
    NW = info.num_cores * info.num_subcores
    idx3 = entry_nodes.reshape(NW, (2 * E) // (NW * _KCH), _KCH)
    n_chunks = (2 * E) // (NW * _KCH)
    n_ph = 5
    idx4 = entry_nodes.reshape(NW, n_ph, n_chunks // n_ph, _KCH)

    gA = _sc_gather(node_rep, idx3)
    gpair = gA.reshape(E, 2 * H)
    epair = edge_rep.reshape(E, 2 * H)

    r2 = lambda x: x.reshape(1, -1)
    s2 = lambda x: x.reshape(1, 1)

    z1, z2, st1, st2 = _tc_pass1(gpair, epair, lvl1_W, lift_W1, s2(eps2))
    P, z3, st3 = _tc_pass2(z1, z2, st1, st2, r2(lvl1_g), r2(lvl1_b),
                           r2(lift_g1), r2(lift_b1), lift_W2, s2(eps1_2))

    zeros = jnp.zeros((N, H), jnp.float32)
    acc = _sc_scatter_add(P.reshape(2 * E, H), idx4, zeros, N)
    acc2 = acc.reshape(2 * N, H)

    node_out = _tc_node(node_rep, acc2, lvl2_W1, r2(lvl2_g1), r2(lvl2_b1),
                        lvl2_W2, r2(lvl2_g2), r2(lvl2_b2), s2(eps1_1))
    edge_out = _tc_pass3(z3, st3, r2(lift_g2), r2(lift_b2)).reshape(2 * E, H)
    return node_out, edge_out


# speedup vs baseline: 1.0778x; 1.0103x over previous
import functools

import jax
import jax.numpy as jnp
from jax import lax
from jax.experimental import pallas as pl
from jax.experimental.pallas import tpu as pltpu
from jax.experimental.pallas import tpu_sc as plsc

_BK = 4000
_KCH = 80


def _sc_gather(table, idx3):
    NW, n_chunks, K = idx3.shape
    D = table.shape[1]
    dt = table.dtype
    total = NW * n_chunks * K
    info = plsc.get_sparse_core_info()
    NC = info.num_cores
    per_w = total // NW
    n2 = n_chunks // 2
    mesh = plsc.VectorSubcoreMesh(core_axis_name="c", subcore_axis_name="s")

    @functools.partial(
        pl.kernel,
        mesh=mesh,
        out_type=jax.ShapeDtypeStruct((total, D), dt),
        scratch_types=[
            pltpu.VMEM((n_chunks, K), jnp.int32),
            pltpu.VMEM((K, D), dt),
            pltpu.VMEM((K, D), dt),
            pltpu.SemaphoreType.DMA,
            pltpu.SemaphoreType.DMA,
            pltpu.SemaphoreType.DMA,
            pltpu.SemaphoreType.DMA,
        ],
    )
    def gather_k(table_hbm, idx_hbm, out_hbm, idx_all, r0, r1,
                 sg0, sg1, so0, so1):
        wid = lax.axis_index("s") * NC + lax.axis_index("c")
        base = wid * per_w
        pltpu.sync_copy(idx_hbm.at[wid], idx_all)

        def start_g(i, buf, sem):
            pltpu.async_copy(table_hbm.at[idx_all.at[i]], buf, sem)

        def wait_g(buf, sem):
            pltpu.make_async_copy(table_hbm.at[idx_all.at[0]], buf, sem).wait()

        def start_o(i, buf, sem):
            off = pl.multiple_of(base + i * K, 8)
            pltpu.async_copy(buf, out_hbm.at[pl.ds(off, K)], sem)

        def wait_o(buf, sem):
            pltpu.make_async_copy(buf, out_hbm.at[pl.ds(base, K)], sem).wait()

        start_g(0, r0, sg0)

        def body(j, carry):
            @pl.when(j > 0)
            def _():
                wait_o(r1, so1)
            start_g(2 * j + 1, r1, sg1)
            wait_g(r0, sg0)
            start_o(2 * j, r0, so0)
            wait_g(r1, sg1)
            wait_o(r0, so0)

            @pl.when(j < n2 - 1)
            def _():
                start_g(2 * j + 2, r0, sg0)
            start_o(2 * j + 1, r1, so1)
            return carry

        lax.fori_loop(0, n2, body, 0)
        wait_o(r1, so1)

    return gather_k(table, idx3)


def _sc_scatter_add(payload, idx4, zeros, n_nodes):
    NW, n_ph, PH, K = idx4.shape
    D = payload.shape[1]
    total = NW * n_ph * PH * K
    n_chunks = n_ph * PH
    info = plsc.get_sparse_core_info()
    NC, NS = info.num_cores, info.num_subcores
    per_w = total // NW
    rows_per_s = (n_nodes // NS) & ~7
    tail = n_nodes - rows_per_s * NS
    mesh = plsc.VectorSubcoreMesh(core_axis_name="c", subcore_axis_name="s")

    @functools.partial(
        pl.kernel,
        mesh=mesh,
        out_type=jax.ShapeDtypeStruct((NC, n_nodes, D), jnp.float32),
        scratch_types=[
            pltpu.VMEM((PH, K), jnp.int32),
            pltpu.VMEM((K, D), jnp.float32),
            pltpu.VMEM((K, D), jnp.float32),
            pltpu.VMEM_SHARED((n_nodes, D), jnp.float32),
            pltpu.SemaphoreType.DMA,
            pltpu.SemaphoreType.DMA,
        ],
    )
    def scatter_k(p_hbm, idx_hbm, z_hbm, out_hbm, idx_ph, r0, r1, acc,
                  sp0, sp1):
        c = lax.axis_index("c")
        s = lax.axis_index("s")

        @pl.when(s == 0)
        def _init():
            pltpu.sync_copy(z_hbm, acc)

        wid = s * NC + c
        base = wid * per_w
        plsc.subcore_barrier()

        def start_p(i, buf, sem):
            off = pl.multiple_of(base + i * K, 8)
            pltpu.async_copy(p_hbm.at[pl.ds(off, K)], buf, sem)

        def wait_p(buf, sem):
            pltpu.make_async_copy(p_hbm.at[pl.ds(base, K)], buf, sem).wait()

        start_p(0, r0, sp0)

        def phase(p, carry):
            pltpu.sync_copy(idx_hbm.at[wid, p], idx_ph)

            def body(jj, c2):
                g0 = p * PH + 2 * jj
                start_p(g0 + 1, r1, sp1)
                wait_p(r0, sp0)
                pltpu.sync_copy(r0, acc.at[idx_ph.at[2 * jj]], add=True)

                @pl.when(g0 + 2 < n_chunks)
                def _():
                    start_p(g0 + 2, r0, sp0)
                wait_p(r1, sp1)
                pltpu.sync_copy(r1, acc.at[idx_ph.at[2 * jj + 1]], add=True)
                return c2

            lax.fori_loop(0, PH // 2, body, carry)
            return carry

        lax.fori_loop(0, n_ph, phase, 0)
        plsc.subcore_barrier()
        r0o = pl.multiple_of(s * rows_per_s, 8)
        pltpu.sync_copy(acc.at[pl.ds(r0o, rows_per_s)],
                        out_hbm.at[c, pl.ds(r0o, rows_per_s)])
        if tail:
            @pl.when(s == NS - 1)
            def _tail():
                t0 = rows_per_s * NS
                pltpu.sync_copy(acc.at[pl.ds(t0, tail)],
                                out_hbm.at[c, pl.ds(t0, tail)])

    return scatter_k(payload, idx4, zeros)


def _tc_pass1(gpair, epair, lvl1_W, lift_W1, eps2):
    Ep, two_h = gpair.shape
    H = two_h // 2
    grid = Ep // _BK
    n_entries = jnp.float32(2 * Ep)
    del n_entries

    def body(g_ref, er_ref, w_ref, u_ref, eps_ref, z1_ref, z2_ref,
             st1_ref, st2_ref):
        f32, bf16 = jnp.float32, jnp.bfloat16
        g = g_ref[...]
        er = er_ref[...]
        ge, go = g[:, :H], g[:, H:]
        ere, ero = er[:, :H], er[:, H:]
        psum = ge + go
        pmean = 0.5 * (ere + ero)
        W = w_ref[...].astype(bf16)
        Wa, Wb, Wc = W[:H], W[H:2 * H], W[2 * H:]

        def bdot(x, w):
            return jnp.dot(x.astype(bf16), w, preferred_element_type=f32)

        sb = bdot(psum, Wb)
        z1e = bdot(ge, Wa) + sb + bdot(ere, Wc)
        z1o = bdot(go, Wa) + sb + bdot(ero, Wc)
        a = 1.0 + eps_ref[0, 0]
        U = u_ref[...].astype(bf16)
        Ua, Ub = U[:H], U[H:]
        tb = bdot(a * pmean + psum, Ub)
        z2e = bdot(a * ere + ge, Ua) + tb
        z2o = bdot(a * ero + go, Ua) + tb
        z1_ref[...] = jnp.concatenate([z1e, z1o], axis=1).astype(bf16)
        z2_ref[...] = jnp.concatenate([z2e, z2o], axis=1).astype(bf16)
        s1 = (jnp.sum(z1e, 0) + jnp.sum(z1o, 0))[None, :]
        q1 = (jnp.sum(z1e * z1e, 0) + jnp.sum(z1o * z1o, 0))[None, :]
        s2 = (jnp.sum(z2e, 0) + jnp.sum(z2o, 0))[None, :]
        q2 = (jnp.sum(z2e * z2e, 0) + jnp.sum(z2o * z2o, 0))[None, :]
        new1 = jnp.concatenate([s1, q1, jnp.zeros((6, H), f32)], axis=0)
        new2 = jnp.concatenate([s2, q2, jnp.zeros((6, 2 * H), f32)], axis=0)
        first = pl.program_id(0) == 0
        st1_ref[...] = jnp.where(first, new1, st1_ref[...] + new1)
        st2_ref[...] = jnp.where(first, new2, st2_ref[...] + new2)

    return pl.pallas_call(
        body,
        grid=(grid,),
        in_specs=[
            pl.BlockSpec((_BK, 2 * H), lambda i: (i, 0)),
            pl.BlockSpec((_BK, 2 * H), lambda i: (i, 0)),
            pl.BlockSpec((3 * H, H), lambda i: (0, 0)),
            pl.BlockSpec((2 * H, 2 * H), lambda i: (0, 0)),
            pl.BlockSpec((1, 1), lambda i: (0, 0)),
        ],
        out_specs=[
            pl.BlockSpec((_BK, 2 * H), lambda i: (i, 0)),
            pl.BlockSpec((_BK, 4 * H), lambda i: (i, 0)),
            pl.BlockSpec((8, H), lambda i: (0, 0)),
            pl.BlockSpec((8, 2 * H), lambda i: (0, 0)),
        ],
        out_shape=[
            jax.ShapeDtypeStruct((Ep, 2 * H), jnp.bfloat16),
            jax.ShapeDtypeStruct((Ep, 4 * H), jnp.bfloat16),
            jax.ShapeDtypeStruct((8, H), jnp.float32),
            jax.ShapeDtypeStruct((8, 2 * H), jnp.float32),
        ],
    )(gpair, epair, lvl1_W, lift_W1, eps2)


def _tc_pass2(z1, z2, st1, st2, lvl1_g, lvl1_b, lift_g1, lift_b1, lift_W2,
              eps1_2):
    Ep, two_h = z1.shape
    H = two_h // 2
    grid = Ep // _BK
    inv_n = 1.0 / float(2 * Ep)

    def body(z1_ref, z2_ref, st1_ref, st2_ref, g1_ref, b1_ref, lg_ref,
             lb_ref, w2_ref, eps_ref, p_ref, z3_ref, st3_ref):
        f32 = jnp.float32
        s1 = st1_ref[0, :]
        q1 = st1_ref[1, :]
        m1 = s1 * inv_n
        v1 = q1 * inv_n - m1 * m1
        sc1 = g1_ref[0, :] * lax.rsqrt(v1 + 1e-5)
        of1 = b1_ref[0, :] - m1 * sc1
        z1b = z1_ref[...].astype(f32)
        h1e = jnp.maximum(z1b[:, :H] * sc1 + of1, 0.0)
        h1o = jnp.maximum(z1b[:, H:] * sc1 + of1, 0.0)
        w = 2.0 + eps_ref[0, 0]
        p_ref[...] = jnp.concatenate([w * h1e + h1o, h1e + w * h1o], axis=1)
        s2 = st2_ref[0, :]
        q2 = st2_ref[1, :]
        m2 = s2 * inv_n
        v2 = q2 * inv_n - m2 * m2
        sc2 = lg_ref[0, :] * lax.rsqrt(v2 + 1e-5)
        of2 = lb_ref[0, :] - m2 * sc2
        z2b = z2_ref[...].astype(f32)
        ue = jnp.maximum(z2b[:, :2 * H] * sc2 + of2, 0.0)
        uo = jnp.maximum(z2b[:, 2 * H:] * sc2 + of2, 0.0)
        W2 = w2_ref[...].astype(jnp.bfloat16)
        z3e = jnp.dot(ue.astype(jnp.bfloat16), W2, preferred_element_type=f32)
        z3o = jnp.dot(uo.astype(jnp.bfloat16), W2, preferred_element_type=f32)
        z3_ref[...] = jnp.concatenate([z3e, z3o], axis=1).astype(jnp.bfloat16)
        s3 = (jnp.sum(z3e, 0) + jnp.sum(z3o, 0))[None, :]
        q3 = (jnp.sum(z3e * z3e, 0) + jnp.sum(z3o * z3o, 0))[None, :]
        new3 = jnp.concatenate([s3, q3, jnp.zeros((6, H), f32)], axis=0)
        first = pl.program_id(0) == 0
        st3_ref[...] = jnp.where(first, new3, st3_ref[...] + new3)

    return pl.pallas_call(
        body,
        grid=(grid,),
        in_specs=[
            pl.BlockSpec((_BK, 2 * H), lambda i: (i, 0)),
            pl.BlockSpec((_BK, 4 * H), lambda i: (i, 0)),
            pl.BlockSpec((8, H), lambda i: (0, 0)),
            pl.BlockSpec((8, 2 * H), lambda i: (0, 0)),
            pl.BlockSpec((1, H), lambda i: (0, 0)),
            pl.BlockSpec((1, H), lambda i: (0, 0)),
            pl.BlockSpec((1, 2 * H), lambda i: (0, 0)),
            pl.BlockSpec((1, 2 * H), lambda i: (0, 0)),
            pl.BlockSpec((2 * H, H), lambda i: (0, 0)),
            pl.BlockSpec((1, 1), lambda i: (0, 0)),
        ],
        out_specs=[
            pl.BlockSpec((_BK, 2 * H), lambda i: (i, 0)),
            pl.BlockSpec((_BK, 2 * H), lambda i: (i, 0)),
            pl.BlockSpec((8, H), lambda i: (0, 0)),
        ],
        out_shape=[
            jax.ShapeDtypeStruct((Ep, 2 * H), jnp.float32),
            jax.ShapeDtypeStruct((Ep, 2 * H), jnp.bfloat16),
            jax.ShapeDtypeStruct((8, H), jnp.float32),
        ],
    )(z1, z2, st1, st2, lvl1_g, lvl1_b, lift_g1, lift_b1, lift_W2, eps1_2)


def _tc_pass3(z3, st3, lift_g2, lift_b2):
    Ep, two_h = z3.shape
    H = two_h // 2
    grid = Ep // _BK
    inv_n = 1.0 / float(2 * Ep)

    def body(z3_ref, st3_ref, g_ref, b_ref, out_ref):
        s3 = st3_ref[0, :]
        q3 = st3_ref[1, :]
        m3 = s3 * inv_n
        v3 = q3 * inv_n - m3 * m3
        sc3 = g_ref[0, :] * lax.rsqrt(v3 + 1e-5)
        of3 = b_ref[0, :] - m3 * sc3
        sc = jnp.concatenate([sc3, sc3])
        of = jnp.concatenate([of3, of3])
        out_ref[...] = jnp.maximum(z3_ref[...].astype(jnp.float32) * sc + of,
                                   0.0)

    return pl.pallas_call(
        body,
        grid=(grid,),
        in_specs=[
            pl.BlockSpec((_BK, 2 * H), lambda i: (i, 0)),
            pl.BlockSpec((8, H), lambda i: (0, 0)),
            pl.BlockSpec((1, H), lambda i: (0, 0)),
            pl.BlockSpec((1, H), lambda i: (0, 0)),
        ],
        out_specs=pl.BlockSpec((_BK, 2 * H), lambda i: (i, 0)),
        out_shape=jax.ShapeDtypeStruct((Ep, 2 * H), jnp.float32),
    )(z3, st3, lift_g2, lift_b2)


def _tc_node(node_rep, acc2, lvl2_W1, lvl2_g1, lvl2_b1, lvl2_W2, lvl2_g2,
             lvl2_b2, eps1_1):
    N, H = node_rep.shape

    def body(nr_ref, acc_ref, w1_ref, g1_ref, b1_ref, w2_ref, g2_ref,
             b2_ref, eps_ref, out_ref):
        f32 = jnp.float32
        acc = acc_ref[...]
        node_in = (1.0 + eps_ref[0, 0]) * nr_ref[...] + acc[:N] + acc[N:]
        z = jnp.dot(node_in, w1_ref[...], preferred_element_type=f32)
        m = jnp.mean(z, axis=0)
        v = jnp.mean(z * z, axis=0) - m * m
        sc = g1_ref[0, :] * lax.rsqrt(v + 1e-5)
        t = jnp.maximum(z * sc + (b1_ref[0, :] - m * sc), 0.0)
        z2 = jnp.dot(t, w2_ref[...], preferred_element_type=f32)
        m2 = jnp.mean(z2, axis=0)
        v2 = jnp.mean(z2 * z2, axis=0) - m2 * m2
        sc2 = g2_ref[0, :] * lax.rsqrt(v2 + 1e-5)
        out_ref[...] = jnp.maximum(z2 * sc2 + (b2_ref[0, :] - m2 * sc2), 0.0)

    return pl.pallas_call(
        body,
        out_shape=jax.ShapeDtypeStruct((N, H), jnp.float32),
    )(node_rep, acc2, lvl2_W1, lvl2_g1, lvl2_b1, lvl2_W2, lvl2_g2, lvl2_b2,
      eps1_1)


def kernel(node_rep, edge_rep, edge_index, lift_W1, lift_g1, lift_b1,
           lift_W2, lift_g2, lift_b2, lvl1_W, lvl1_g, lvl1_b, lvl2_W1,
           lvl2_g1, lvl2_b1, lvl2_W2, lvl2_g2, lvl2_b2, eps1_1, eps1_2,
           eps2):
    N, H = node_rep.shape
    E = edge_index.shape[1]
    entry_nodes = edge_index.T.reshape(-1)
    info = plsc.get_sparse_core_info()
    NW = info.num_cores * info.num_subcores
    idx3 = entry_nodes.reshape(NW, (2 * E) // (NW * _KCH), _KCH)
    n_chunks = (2 * E) // (NW * _KCH)
    n_ph = 5
    idx4 = entry_nodes.reshape(NW, n_ph, n_chunks // n_ph, _KCH)

    gA = _sc_gather(node_rep, idx3)
    gpair = gA.reshape(E, 2 * H)
    epair = edge_rep.reshape(E, 2 * H)

    r2 = lambda x: x.reshape(1, -1)
    s2 = lambda x: x.reshape(1, 1)

    z1, z2, st1, st2 = _tc_pass1(gpair, epair, lvl1_W, lift_W1, s2(eps2))
    P, z3, st3 = _tc_pass2(z1, z2, st1, st2, r2(lvl1_g), r2(lvl1_b),
                           r2(lift_g1), r2(lift_b1), lift_W2, s2(eps1_2))

    zeros = jnp.zeros((N, H), jnp.float32)
    acc = _sc_scatter_add(P.reshape(2 * E, H), idx4, zeros, N)
    acc2 = acc.reshape(2 * N, H)

    node_out = _tc_node(node_rep, acc2, lvl2_W1, r2(lvl2_g1), r2(lvl2_b1),
                        lvl2_W2, r2(lvl2_g2), r2(lvl2_b2), s2(eps1_1))
    edge_out = _tc_pass3(z3, st3, r2(lift_g2), r2(lift_b2)).reshape(2 * E, H)
    return node_out, edge_out


# speedup vs baseline: 1.0780x; 1.0002x over previous
import functools

import jax
import jax.numpy as jnp
from jax import lax
from jax.experimental import pallas as pl
from jax.experimental.pallas import tpu as pltpu
from jax.experimental.pallas import tpu_sc as plsc

_BK = 5000
_KCH = 80


def _sc_gather(table, idx3):
    NW, n_chunks, K = idx3.shape
    D = table.shape[1]
    dt = table.dtype
    total = NW * n_chunks * K
    info = plsc.get_sparse_core_info()
    NC = info.num_cores
    per_w = total // NW
    n2 = n_chunks // 2
    mesh = plsc.VectorSubcoreMesh(core_axis_name="c", subcore_axis_name="s")

    @functools.partial(
        pl.kernel,
        mesh=mesh,
        out_type=jax.ShapeDtypeStruct((total, D), dt),
        scratch_types=[
            pltpu.VMEM((n_chunks, K), jnp.int32),
            pltpu.VMEM((K, D), dt),
            pltpu.VMEM((K, D), dt),
            pltpu.SemaphoreType.DMA,
            pltpu.SemaphoreType.DMA,
            pltpu.SemaphoreType.DMA,
            pltpu.SemaphoreType.DMA,
        ],
    )
    def gather_k(table_hbm, idx_hbm, out_hbm, idx_all, r0, r1,
                 sg0, sg1, so0, so1):
        wid = lax.axis_index("s") * NC + lax.axis_index("c")
        base = wid * per_w
        pltpu.sync_copy(idx_hbm.at[wid], idx_all)

        def start_g(i, buf, sem):
            pltpu.async_copy(table_hbm.at[idx_all.at[i]], buf, sem)

        def wait_g(buf, sem):
            pltpu.make_async_copy(table_hbm.at[idx_all.at[0]], buf, sem).wait()

        def start_o(i, buf, sem):
            off = pl.multiple_of(base + i * K, 8)
            pltpu.async_copy(buf, out_hbm.at[pl.ds(off, K)], sem)

        def wait_o(buf, sem):
            pltpu.make_async_copy(buf, out_hbm.at[pl.ds(base, K)], sem).wait()

        start_g(0, r0, sg0)

        def body(j, carry):
            @pl.when(j > 0)
            def _():
                wait_o(r1, so1)
            start_g(2 * j + 1, r1, sg1)
            wait_g(r0, sg0)
            start_o(2 * j, r0, so0)
            wait_g(r1, sg1)
            wait_o(r0, so0)

            @pl.when(j < n2 - 1)
            def _():
                start_g(2 * j + 2, r0, sg0)
            start_o(2 * j + 1, r1, so1)
            return carry

        lax.fori_loop(0, n2, body, 0)
        wait_o(r1, so1)

    return gather_k(table, idx3)


def _sc_scatter_add(payload, idx4, zeros, n_nodes):
    NW, n_ph, PH, K = idx4.shape
    D = payload.shape[1]
    total = NW * n_ph * PH * K
    n_chunks = n_ph * PH
    info = plsc.get_sparse_core_info()
    NC, NS = info.num_cores, info.num_subcores
    per_w = total // NW
    rows_per_s = (n_nodes // NS) & ~7
    tail = n_nodes - rows_per_s * NS
    mesh = plsc.VectorSubcoreMesh(core_axis_name="c", subcore_axis_name="s")

    @functools.partial(
        pl.kernel,
        mesh=mesh,
        out_type=jax.ShapeDtypeStruct((NC, n_nodes, D), jnp.float32),
        scratch_types=[
            pltpu.VMEM((PH, K), jnp.int32),
            pltpu.VMEM((K, D), jnp.float32),
            pltpu.VMEM((K, D), jnp.float32),
            pltpu.VMEM_SHARED((n_nodes, D), jnp.float32),
            pltpu.SemaphoreType.DMA,
            pltpu.SemaphoreType.DMA,
        ],
    )
    def scatter_k(p_hbm, idx_hbm, z_hbm, out_hbm, idx_ph, r0, r1, acc,
                  sp0, sp1):
        c = lax.axis_index("c")
        s = lax.axis_index("s")

        @pl.when(s == 0)
        def _init():
            pltpu.sync_copy(z_hbm, acc)

        wid = s * NC + c
        base = wid * per_w
        plsc.subcore_barrier()

        def start_p(i, buf, sem):
            off = pl.multiple_of(base + i * K, 8)
            pltpu.async_copy(p_hbm.at[pl.ds(off, K)], buf, sem)

        def wait_p(buf, sem):
            pltpu.make_async_copy(p_hbm.at[pl.ds(base, K)], buf, sem).wait()

        start_p(0, r0, sp0)

        def phase(p, carry):
            pltpu.sync_copy(idx_hbm.at[wid, p], idx_ph)

            def body(jj, c2):
                g0 = p * PH + 2 * jj
                start_p(g0 + 1, r1, sp1)
                wait_p(r0, sp0)
                pltpu.sync_copy(r0, acc.at[idx_ph.at[2 * jj]], add=True)

                @pl.when(g0 + 2 < n_chunks)
                def _():
                    start_p(g0 + 2, r0, sp0)
                wait_p(r1, sp1)
                pltpu.sync_copy(r1, acc.at[idx_ph.at[2 * jj + 1]], add=True)
                return c2

            lax.fori_loop(0, PH // 2, body, carry)
            return carry

        lax.fori_loop(0, n_ph, phase, 0)
        plsc.subcore_barrier()
        r0o = pl.multiple_of(s * rows_per_s, 8)
        pltpu.sync_copy(acc.at[pl.ds(r0o, rows_per_s)],
                        out_hbm.at[c, pl.ds(r0o, rows_per_s)])
        if tail:
            @pl.when(s == NS - 1)
            def _tail():
                t0 = rows_per_s * NS
                pltpu.sync_copy(acc.at[pl.ds(t0, tail)],
                                out_hbm.at[c, pl.ds(t0, tail)])

    return scatter_k(payload, idx4, zeros)


def _tc_pass1(gpair, epair, lvl1_W, lift_W1, eps2):
    Ep, two_h = gpair.shape
    H = two_h // 2
    grid = Ep // _BK
    n_entries = jnp.float32(2 * Ep)
    del n_entries

    def body(g_ref, er_ref, w_ref, u_ref, eps_ref, z1_ref, z2_ref,
             st1_ref, st2_ref):
        f32, bf16 = jnp.float32, jnp.bfloat16
        g = g_ref[...]
        er = er_ref[...]
        ge, go = g[:, :H], g[:, H:]
        ere, ero = er[:, :H], er[:, H:]
        psum = ge + go
        pmean = 0.5 * (ere + ero)
        W = w_ref[...].astype(bf16)
        Wa, Wb, Wc = W[:H], W[H:2 * H], W[2 * H:]

        def bdot(x, w):
            return jnp.dot(x.astype(bf16), w, preferred_element_type=f32)

        sb = bdot(psum, Wb)
        z1e = bdot(ge, Wa) + sb + bdot(ere, Wc)
        z1o = bdot(go, Wa) + sb + bdot(ero, Wc)
        a = 1.0 + eps_ref[0, 0]
        U = u_ref[...].astype(bf16)
        Ua, Ub = U[:H], U[H:]
        tb = bdot(a * pmean + psum, Ub)
        z2e = bdot(a * ere + ge, Ua) + tb
        z2o = bdot(a * ero + go, Ua) + tb
        z1_ref[...] = jnp.concatenate([z1e, z1o], axis=1).astype(bf16)
        z2_ref[...] = jnp.concatenate([z2e, z2o], axis=1).astype(bf16)
        s1 = (jnp.sum(z1e, 0) + jnp.sum(z1o, 0))[None, :]
        q1 = (jnp.sum(z1e * z1e, 0) + jnp.sum(z1o * z1o, 0))[None, :]
        s2 = (jnp.sum(z2e, 0) + jnp.sum(z2o, 0))[None, :]
        q2 = (jnp.sum(z2e * z2e, 0) + jnp.sum(z2o * z2o, 0))[None, :]
        new1 = jnp.concatenate([s1, q1, jnp.zeros((6, H), f32)], axis=0)
        new2 = jnp.concatenate([s2, q2, jnp.zeros((6, 2 * H), f32)], axis=0)
        first = pl.program_id(0) == 0
        st1_ref[...] = jnp.where(first, new1, st1_ref[...] + new1)
        st2_ref[...] = jnp.where(first, new2, st2_ref[...] + new2)

    return pl.pallas_call(
        body,
        grid=(grid,),
        in_specs=[
            pl.BlockSpec((_BK, 2 * H), lambda i: (i, 0)),
            pl.BlockSpec((_BK, 2 * H), lambda i: (i, 0)),
            pl.BlockSpec((3 * H, H), lambda i: (0, 0)),
            pl.BlockSpec((2 * H, 2 * H), lambda i: (0, 0)),
            pl.BlockSpec((1, 1), lambda i: (0, 0)),
        ],
        out_specs=[
            pl.BlockSpec((_BK, 2 * H), lambda i: (i, 0)),
            pl.BlockSpec((_BK, 4 * H), lambda i: (i, 0)),
            pl.BlockSpec((8, H), lambda i: (0, 0)),
            pl.BlockSpec((8, 2 * H), lambda i: (0, 0)),
        ],
        out_shape=[
            jax.ShapeDtypeStruct((Ep, 2 * H), jnp.bfloat16),
            jax.ShapeDtypeStruct((Ep, 4 * H), jnp.bfloat16),
            jax.ShapeDtypeStruct((8, H), jnp.float32),
            jax.ShapeDtypeStruct((8, 2 * H), jnp.float32),
        ],
    )(gpair, epair, lvl1_W, lift_W1, eps2)


def _tc_pass2(z1, z2, st1, st2, lvl1_g, lvl1_b, lift_g1, lift_b1, lift_W2,
              eps1_2):
    Ep, two_h = z1.shape
    H = two_h // 2
    grid = Ep // _BK
    inv_n = 1.0 / float(2 * Ep)

    def body(z1_ref, z2_ref, st1_ref, st2_ref, g1_ref, b1_ref, lg_ref,
             lb_ref, w2_ref, eps_ref, p_ref, z3_ref, st3_ref):
        f32 = jnp.float32
        s1 = st1_ref[0, :]
        q1 = st1_ref[1, :]
        m1 = s1 * inv_n
        v1 = q1 * inv_n - m1 * m1
        sc1 = g1_ref[0, :] * lax.rsqrt(v1 + 1e-5)
        of1 = b1_ref[0, :] - m1 * sc1
        z1b = z1_ref[...].astype(f32)
        h1e = jnp.maximum(z1b[:, :H] * sc1 + of1, 0.0)
        h1o = jnp.maximum(z1b[:, H:] * sc1 + of1, 0.0)
        w = 2.0 + eps_ref[0, 0]
        p_ref[...] = jnp.concatenate([w * h1e + h1o, h1e + w * h1o], axis=1)
        s2 = st2_ref[0, :]
        q2 = st2_ref[1, :]
        m2 = s2 * inv_n
        v2 = q2 * inv_n - m2 * m2
        sc2 = lg_ref[0, :] * lax.rsqrt(v2 + 1e-5)
        of2 = lb_ref[0, :] - m2 * sc2
        z2b = z2_ref[...].astype(f32)
        ue = jnp.maximum(z2b[:, :2 * H] * sc2 + of2, 0.0)
        uo = jnp.maximum(z2b[:, 2 * H:] * sc2 + of2, 0.0)
        W2 = w2_ref[...].astype(jnp.bfloat16)
        z3e = jnp.dot(ue.astype(jnp.bfloat16), W2, preferred_element_type=f32)
        z3o = jnp.dot(uo.astype(jnp.bfloat16), W2, preferred_element_type=f32)
        z3_ref[...] = jnp.concatenate([z3e, z3o], axis=1).astype(jnp.bfloat16)
        s3 = (jnp.sum(z3e, 0) + jnp.sum(z3o, 0))[None, :]
        q3 = (jnp.sum(z3e * z3e, 0) + jnp.sum(z3o * z3o, 0))[None, :]
        new3 = jnp.concatenate([s3, q3, jnp.zeros((6, H), f32)], axis=0)
        first = pl.program_id(0) == 0
        st3_ref[...] = jnp.where(first, new3, st3_ref[...] + new3)

    return pl.pallas_call(
        body,
        grid=(grid,),
        in_specs=[
            pl.BlockSpec((_BK, 2 * H), lambda i: (i, 0)),
            pl.BlockSpec((_BK, 4 * H), lambda i: (i, 0)),
            pl.BlockSpec((8, H), lambda i: (0, 0)),
            pl.BlockSpec((8, 2 * H), lambda i: (0, 0)),
            pl.BlockSpec((1, H), lambda i: (0, 0)),
            pl.BlockSpec((1, H), lambda i: (0, 0)),
            pl.BlockSpec((1, 2 * H), lambda i: (0, 0)),
            pl.BlockSpec((1, 2 * H), lambda i: (0, 0)),
            pl.BlockSpec((2 * H, H), lambda i: (0, 0)),
            pl.BlockSpec((1, 1), lambda i: (0, 0)),
        ],
        out_specs=[
            pl.BlockSpec((_BK, 2 * H), lambda i: (i, 0)),
            pl.BlockSpec((_BK, 2 * H), lambda i: (i, 0)),
            pl.BlockSpec((8, H), lambda i: (0, 0)),
        ],
        out_shape=[
            jax.ShapeDtypeStruct((Ep, 2 * H), jnp.float32),
            jax.ShapeDtypeStruct((Ep, 2 * H), jnp.bfloat16),
            jax.ShapeDtypeStruct((8, H), jnp.float32),
        ],
    )(z1, z2, st1, st2, lvl1_g, lvl1_b, lift_g1, lift_b1, lift_W2, eps1_2)


def _tc_pass3(z3, st3, lift_g2, lift_b2):
    Ep, two_h = z3.shape
    H = two_h // 2
    grid = Ep // _BK
    inv_n = 1.0 / float(2 * Ep)

    def body(z3_ref, st3_ref, g_ref, b_ref, out_ref):
        s3 = st3_ref[0, :]
        q3 = st3_ref[1, :]
        m3 = s3 * inv_n
        v3 = q3 * inv_n - m3 * m3
        sc3 = g_ref[0, :] * lax.rsqrt(v3 + 1e-5)
        of3 = b_ref[0, :] - m3 * sc3
        sc = jnp.concatenate([sc3, sc3])
        of = jnp.concatenate([of3, of3])
        out_ref[...] = jnp.maximum(z3_ref[...].astype(jnp.float32) * sc + of,
                                   0.0)

    return pl.pallas_call(
        body,
        grid=(grid,),
        in_specs=[
            pl.BlockSpec((_BK, 2 * H), lambda i: (i, 0)),
            pl.BlockSpec((8, H), lambda i: (0, 0)),
            pl.BlockSpec((1, H), lambda i: (0, 0)),
            pl.BlockSpec((1, H), lambda i: (0, 0)),
        ],
        out_specs=pl.BlockSpec((_BK, 2 * H), lambda i: (i, 0)),
        out_shape=jax.ShapeDtypeStruct((Ep, 2 * H), jnp.float32),
    )(z3, st3, lift_g2, lift_b2)


def _tc_node(node_rep, acc2, lvl2_W1, lvl2_g1, lvl2_b1, lvl2_W2, lvl2_g2,
             lvl2_b2, eps1_1):
    N, H = node_rep.shape

    def body(nr_ref, acc_ref, w1_ref, g1_ref, b1_ref, w2_ref, g2_ref,
             b2_ref, eps_ref, out_ref):
        f32 = jnp.float32
        acc = acc_ref[...]
        node_in = (1.0 + eps_ref[0, 0]) * nr_ref[...] + acc[:N] + acc[N:]
        z = jnp.dot(node_in, w1_ref[...], preferred_element_type=f32)
        m = jnp.mean(z, axis=0)
        v = jnp.mean(z * z, axis=0) - m * m
        sc = g1_ref[0, :] * lax.rsqrt(v + 1e-5)
        t = jnp.maximum(z * sc + (b1_ref[0, :] - m * sc), 0.0)
        z2 = jnp.dot(t, w2_ref[...], preferred_element_type=f32)
        m2 = jnp.mean(z2, axis=0)
        v2 = jnp.mean(z2 * z2, axis=0) - m2 * m2
        sc2 = g2_ref[0, :] * lax.rsqrt(v2 + 1e-5)
        out_ref[...] = jnp.maximum(z2 * sc2 + (b2_ref[0, :] - m2 * sc2), 0.0)

    return pl.pallas_call(
        body,
        out_shape=jax.ShapeDtypeStruct((N, H), jnp.float32),
    )(node_rep, acc2, lvl2_W1, lvl2_g1, lvl2_b1, lvl2_W2, lvl2_g2, lvl2_b2,
      eps1_1)


def kernel(node_rep, edge_rep, edge_index, lift_W1, lift_g1, lift_b1,
           lift_W2, lift_g2, lift_b2, lvl1_W, lvl1_g, lvl1_b, lvl2_W1,
           lvl2_g1, lvl2_b1, lvl2_W2, lvl2_g2, lvl2_b2, eps1_1, eps1_2,
           eps2):
    N, H = node_rep.shape
    E = edge_index.shape[1]
    entry_nodes = edge_index.T.reshape(-1)
    info = plsc.get_sparse_core_info()
    NW = info.num_cores * info.num_subcores
    idx3 = entry_nodes.reshape(NW, (2 * E) // (NW * _KCH), _KCH)
    n_chunks = (2 * E) // (NW * _KCH)
    n_ph = 5
    idx4 = entry_nodes.reshape(NW, n_ph, n_chunks // n_ph, _KCH)

    gA = _sc_gather(node_rep, idx3)
    gpair = gA.reshape(E, 2 * H)
    epair = edge_rep.reshape(E, 2 * H)

    r2 = lambda x: x.reshape(1, -1)
    s2 = lambda x: x.reshape(1, 1)

    z1, z2, st1, st2 = _tc_pass1(gpair, epair, lvl1_W, lift_W1, s2(eps2))
    P, z3, st3 = _tc_pass2(z1, z2, st1, st2, r2(lvl1_g), r2(lvl1_b),
                           r2(lift_g1), r2(lift_b1), lift_W2, s2(eps1_2))

    zeros = jnp.zeros((N, H), jnp.float32)
    acc = _sc_scatter_add(P.reshape(2 * E, H), idx4, zeros, N)
    acc2 = acc.reshape(2 * N, H)

    node_out = _tc_node(node_rep, acc2, lvl2_W1, r2(lvl2_g1), r2(lvl2_b1),
                        lvl2_W2, r2(lvl2_g2), r2(lvl2_b2), s2(eps1_1))
    edge_out = _tc_pass3(z3, st3, r2(lift_g2), r2(lift_b2)).reshape(2 * E, H)
    return node_out, edge_out
